# baseline (speedup 1.0000x reference)
import functools

import jax
import jax.numpy as jnp
from jax import lax
from jax.experimental import pallas as pl
from jax.experimental.pallas import tpu as pltpu
from jax.experimental.pallas import tpu_sc as plsc

NUM_USERS = 100000
MAX_MEM = 16
D_KEY = 64
D_VALUE = 64
TOP_K = 4
INV_TEMP = 10.0
BATCH = 4096

NC, NS, LANES = 2, 16, 16
NW = NC * NS
BPW = BATCH // NW
CHUNK = 16
NCHUNK = BPW // CHUNK
ROW_W = MAX_MEM * D_KEY


def _tc_project_normalize(query, W_key):
    def body(x_ref, w_ref, o_ref):
        q = lax.dot_general(x_ref[...], w_ref[...], (((1,), (1,)), ((), ())),
                            preferred_element_type=jnp.float32)
        n = jnp.sqrt(jnp.sum(q * q, axis=-1, keepdims=True))
        o_ref[...] = q / jnp.maximum(n, 1e-12)

    grid = 8
    blk = BATCH // grid
    return pl.pallas_call(
        body,
        grid=(grid,),
        in_specs=[pl.BlockSpec((blk, D_KEY), lambda i: (i, 0)),
                  pl.BlockSpec((D_KEY, D_KEY), lambda i: (0, 0))],
        out_specs=pl.BlockSpec((blk, D_KEY), lambda i: (i, 0)),
        out_shape=jax.ShapeDtypeStruct((BATCH, D_KEY), jnp.float32),
    )(query, W_key)


def _tc_out_project(blended, W_scaled):
    def body(x_ref, w_ref, o_ref):
        o_ref[...] = lax.dot_general(x_ref[...], w_ref[...],
                                     (((1,), (1,)), ((), ())),
                                     preferred_element_type=jnp.float32)

    grid = 8
    blk = BATCH // grid
    return pl.pallas_call(
        body,
        grid=(grid,),
        in_specs=[pl.BlockSpec((blk, D_KEY), lambda i: (i, 0)),
                  pl.BlockSpec((D_VALUE, D_KEY), lambda i: (0, 0))],
        out_specs=pl.BlockSpec((blk, D_VALUE), lambda i: (i, 0)),
        out_shape=jax.ShapeDtypeStruct((BATCH, D_VALUE), jnp.float32),
    )(blended, W_scaled)


def _newton_rsqrt(x):
    i = plsc.bitcast(x, jnp.int32)
    i = jnp.int32(0x5F3759DF) - lax.shift_right_logical(i, 1)
    y = plsc.bitcast(i, jnp.float32)
    for _ in range(3):
        y = y * (1.5 - 0.5 * x * y * y)
    return y


def _sc_body(keys_hbm, values_hbm, qn_hbm, uid_hbm, mc_hbm, out_hbm,
             idx_v, counts_v, qn_v, kbuf0, vbuf0, kbuf1, vbuf1, out_v, sem):
    wid = lax.axis_index("s") * NC + lax.axis_index("c")
    base = wid * BPW
    iota = lax.broadcasted_iota(jnp.int32, (LANES,), 0)

    pltpu.sync_copy(uid_hbm.at[pl.ds(base, BPW)], idx_v)
    cdesc = pltpu.async_copy(mc_hbm.at[idx_v], counts_v, sem)
    pltpu.sync_copy(qn_hbm.at[pl.ds(base, BPW)], qn_v)
    cdesc.wait()

    bufs = ((kbuf0, vbuf0), (kbuf1, vbuf1))

    def start_chunk(c):
        cidx = idx_v[pl.ds(c * CHUNK, CHUNK)]
        kb, vb = bufs[c % 2]
        return (pltpu.async_copy(keys_hbm.at[cidx], kb, sem),
                pltpu.async_copy(values_hbm.at[cidx], vb, sem))

    descs = [start_chunk(0), None]
    for c in range(NCHUNK):
        if c + 1 < NCHUNK:
            descs[(c + 1) % 2] = start_chunk(c + 1)
        d1, d2 = descs[c % 2]
        d1.wait()
        d2.wait()
        kbuf, vbuf = bufs[c % 2]

        def do_row(r, _, c=c, kbuf=kbuf, vbuf=vbuf):
            g = c * CHUNK + r
            row_idx = jnp.full((LANES,), r, jnp.int32)
            col_base = iota * D_KEY

            zero = jnp.zeros((LANES,), jnp.float32)
            acc = zero
            nrm = zero
            for j in range(D_KEY // LANES):
                qvec = qn_v[g, pl.ds(j * LANES, LANES)]
                for t in range(LANES):
                    d = j * LANES + t
                    col = plsc.load_gather(
                        kbuf, [row_idx, iota, jnp.full((LANES,), d, jnp.int32)])
                    acc = acc + col * qvec[t]
                    nrm = nrm + col * col

            inv = _newton_rsqrt(jnp.maximum(nrm, 1e-24))
            sims = acc * inv
            cnt = plsc.load_gather(counts_v, [jnp.full((LANES,), g, jnp.int32)])
            sims = jnp.where(iota < cnt, sims, -1e9)
            svals, sids = plsc.sort_key_val(sims, iota, descending=True)
            smax = jnp.max(sims)
            e = jnp.exp((svals - smax) * INV_TEMP)
            e = jnp.where(iota < TOP_K, e, 0.0)
            w = e / jnp.sum(e)

            outs = [zero] * (D_KEY // LANES)
            for k in range(TOP_K):
                wk = w[k]
                slot = jnp.full((LANES,), sids[k], jnp.int32)
                for j in range(D_KEY // LANES):
                    vrow = plsc.load_gather(
                        vbuf, [row_idx, slot, j * LANES + iota])
                    outs[j] = outs[j] + wk * vrow
            for j in range(D_KEY // LANES):
                out_v[g, pl.ds(j * LANES, LANES)] = outs[j]
            return 0

        lax.fori_loop(0, CHUNK, do_row, 0)

    pltpu.sync_copy(out_v, out_hbm.at[pl.ds(base, BPW)])


def _sc_retrieve(kflat, vflat, qn, user_ids, memory_count):
    mesh = plsc.VectorSubcoreMesh(core_axis_name="c", subcore_axis_name="s")
    f = pl.kernel(
        _sc_body,
        out_type=jax.ShapeDtypeStruct((BATCH, D_KEY), jnp.float32),
        mesh=mesh,
        scratch_types=[
            pltpu.VMEM((BPW,), jnp.int32),
            pltpu.VMEM((BPW,), jnp.int32),
            pltpu.VMEM((BPW, D_KEY), jnp.float32),
            pltpu.VMEM((CHUNK, MAX_MEM, D_KEY), jnp.float32),
            pltpu.VMEM((CHUNK, MAX_MEM, D_KEY), jnp.float32),
            pltpu.VMEM((CHUNK, MAX_MEM, D_KEY), jnp.float32),
            pltpu.VMEM((CHUNK, MAX_MEM, D_KEY), jnp.float32),
            pltpu.VMEM((BPW, D_KEY), jnp.float32),
            pltpu.SemaphoreType.DMA,
        ],
        compiler_params=pltpu.CompilerParams(use_tc_tiling_on_sc=False,
                                             needs_layout_passes=False),
    )
    return f(kflat, vflat, qn, user_ids, memory_count)


def kernel(query, keys_buf, values_buf, W_key, W_val, episodic_scale,
           user_ids, memory_count):
    qn = _tc_project_normalize(query, W_key)
    blended = _sc_retrieve(keys_buf, values_buf, qn,
                           user_ids.astype(jnp.int32),
                           memory_count.astype(jnp.int32))
    return _tc_out_project(blended, W_val * episodic_scale)


# speedup vs baseline: 1.5024x; 1.5024x over previous
import functools

import jax
import jax.numpy as jnp
from jax import lax
from jax.experimental import pallas as pl
from jax.experimental.pallas import tpu as pltpu
from jax.experimental.pallas import tpu_sc as plsc

NUM_USERS = 100000
MAX_MEM = 16
D_KEY = 64
D_VALUE = 64
TOP_K = 4
INV_TEMP = 10.0
BATCH = 4096

NC, NS, LANES = 2, 16, 16
NW = NC * NS
BPW = BATCH // NW
CHUNK = 16
NCHUNK = BPW // CHUNK
ROW_W = MAX_MEM * D_KEY


def _tc_project_normalize(query, W_key):
    def body(x_ref, w_ref, o_ref):
        q = lax.dot_general(x_ref[...], w_ref[...], (((1,), (1,)), ((), ())),
                            preferred_element_type=jnp.float32)
        n = jnp.sqrt(jnp.sum(q * q, axis=-1, keepdims=True))
        o_ref[...] = q / jnp.maximum(n, 1e-12)

    grid = 8
    blk = BATCH // grid
    return pl.pallas_call(
        body,
        grid=(grid,),
        in_specs=[pl.BlockSpec((blk, D_KEY), lambda i: (i, 0)),
                  pl.BlockSpec((D_KEY, D_KEY), lambda i: (0, 0))],
        out_specs=pl.BlockSpec((blk, D_KEY), lambda i: (i, 0)),
        out_shape=jax.ShapeDtypeStruct((BATCH, D_KEY), jnp.float32),
    )(query, W_key)


def _tc_out_project(blended, W_scaled):
    def body(x_ref, w_ref, o_ref):
        o_ref[...] = lax.dot_general(x_ref[...], w_ref[...],
                                     (((1,), (1,)), ((), ())),
                                     preferred_element_type=jnp.float32)

    grid = 8
    blk = BATCH // grid
    return pl.pallas_call(
        body,
        grid=(grid,),
        in_specs=[pl.BlockSpec((blk, D_KEY), lambda i: (i, 0)),
                  pl.BlockSpec((D_VALUE, D_KEY), lambda i: (0, 0))],
        out_specs=pl.BlockSpec((blk, D_VALUE), lambda i: (i, 0)),
        out_shape=jax.ShapeDtypeStruct((BATCH, D_VALUE), jnp.float32),
    )(blended, W_scaled)


def _newton_rsqrt(x):
    i = plsc.bitcast(x, jnp.int32)
    i = jnp.int32(0x5F3759DF) - lax.shift_right_logical(i, 1)
    y = plsc.bitcast(i, jnp.float32)
    for _ in range(3):
        y = y * (1.5 - 0.5 * x * y * y)
    return y


def _sc_body(keys_hbm, values_hbm, qn_hbm, uid_hbm, mc_hbm, out_hbm,
             idx_v, counts_v, qn_v, kbuf0, vbuf0, kbuf1, vbuf1, out_v, sem):
    wid = lax.axis_index("s") * NC + lax.axis_index("c")
    base = wid * BPW
    iota = lax.broadcasted_iota(jnp.int32, (LANES,), 0)

    pltpu.sync_copy(uid_hbm.at[pl.ds(base, BPW)], idx_v)
    cdesc = pltpu.async_copy(mc_hbm.at[idx_v], counts_v, sem)
    pltpu.sync_copy(qn_hbm.at[pl.ds(base, BPW)], qn_v)
    cdesc.wait()

    bufs = ((kbuf0, vbuf0), (kbuf1, vbuf1))

    def start_chunk(c):
        cidx = idx_v[pl.ds(c * CHUNK, CHUNK)]
        kb, vb = bufs[c % 2]
        return [pltpu.async_copy(keys_hbm.at[cidx], kb, sem),
                pltpu.async_copy(values_hbm.at[cidx], vb, sem)]

    descs = [start_chunk(0), None]
    for c in range(NCHUNK):
        if c + 1 < NCHUNK:
            descs[(c + 1) % 2] = start_chunk(c + 1)
        for d in descs[c % 2]:
            d.wait()
        kbuf, vbuf = bufs[c % 2]

        @plsc.parallel_loop(0, CHUNK, 1)
        def do_row(r, c=c, kbuf=kbuf, vbuf=vbuf):
            g = c * CHUNK + r
            row_idx = jnp.full((LANES,), r, jnp.int32)
            col_base = iota * D_KEY

            zero = jnp.zeros((LANES,), jnp.float32)
            acc = zero
            nrm = zero
            for j in range(D_KEY // LANES):
                qvec = qn_v[g, pl.ds(j * LANES, LANES)]
                for t in range(LANES):
                    d = j * LANES + t
                    col = plsc.load_gather(kbuf, [row_idx, col_base + d])
                    acc = acc + col * qvec[t]
                    nrm = nrm + col * col

            inv = _newton_rsqrt(jnp.maximum(nrm, 1e-24))
            sims = acc * inv
            cnt = plsc.load_gather(counts_v, [jnp.full((LANES,), g, jnp.int32)])
            sims = jnp.where(iota < cnt, sims, -1e9)
            svals, sids = plsc.sort_key_val(sims, iota, descending=True)
            smax = jnp.max(sims)
            e = jnp.exp((svals - smax) * INV_TEMP)
            e = jnp.where(iota < TOP_K, e, 0.0)
            w = e / jnp.sum(e)

            outs = [zero] * (D_KEY // LANES)
            for k in range(TOP_K):
                wk = w[k]
                col0 = sids[k] * D_KEY
                for j in range(D_KEY // LANES):
                    vrow = plsc.load_gather(
                        vbuf, [row_idx, col0 + j * LANES + iota])
                    outs[j] = outs[j] + wk * vrow
            for j in range(D_KEY // LANES):
                out_v[g, pl.ds(j * LANES, LANES)] = outs[j]

    pltpu.sync_copy(out_v, out_hbm.at[pl.ds(base, BPW)])


def _sc_retrieve(kflat, vflat, qn, user_ids, memory_count):
    mesh = plsc.VectorSubcoreMesh(core_axis_name="c", subcore_axis_name="s")
    f = pl.kernel(
        _sc_body,
        out_type=jax.ShapeDtypeStruct((BATCH, D_KEY), jnp.float32),
        mesh=mesh,
        scratch_types=[
            pltpu.VMEM((BPW,), jnp.int32),
            pltpu.VMEM((BPW,), jnp.int32),
            pltpu.VMEM((BPW, D_KEY), jnp.float32),
            pltpu.VMEM((CHUNK, ROW_W), jnp.float32),
            pltpu.VMEM((CHUNK, ROW_W), jnp.float32),
            pltpu.VMEM((CHUNK, ROW_W), jnp.float32),
            pltpu.VMEM((CHUNK, ROW_W), jnp.float32),
            pltpu.VMEM((BPW, D_KEY), jnp.float32),
            pltpu.SemaphoreType.DMA,
        ],
        compiler_params=pltpu.CompilerParams(use_tc_tiling_on_sc=False,
                                             needs_layout_passes=False),
    )
    return f(kflat, vflat, qn, user_ids, memory_count)


def kernel(query, keys_buf, values_buf, W_key, W_val, episodic_scale,
           user_ids, memory_count):
    kflat = keys_buf.reshape(NUM_USERS, ROW_W)
    vflat = values_buf.reshape(NUM_USERS, ROW_W)
    qn = _tc_project_normalize(query, W_key)
    blended = _sc_retrieve(kflat, vflat, qn,
                           user_ids.astype(jnp.int32),
                           memory_count.astype(jnp.int32))
    return _tc_out_project(blended, W_val * episodic_scale)
